# baseline (speedup 1.0000x reference)
import jax
import jax.numpy as jnp
from jax import lax
from jax.experimental import pallas as pl
from jax.experimental.pallas import tpu as pltpu
from jax.experimental.pallas import tpu_sc as plsc

_NC = 2
_NS = 16
_NW = _NC * _NS
_L = 16

_GE = 80
_CG = 5
_CE = _GE * _CG


def _proj_body(mvc_ref, w_ref, b_ref, out_ref):
    out_ref[...] = (
        jnp.dot(mvc_ref[...], w_ref[...], preferred_element_type=jnp.float32)
        + b_ref[...]
    )


def _node_projection(mvc, wcat, bcat):
    n, d = mvc.shape
    w = wcat.shape[1]
    blk = 1000
    return pl.pallas_call(
        _proj_body,
        grid=(n // blk,),
        in_specs=[
            pl.BlockSpec((blk, d), lambda i: (i, 0)),
            pl.BlockSpec((d, w), lambda i: (0, 0)),
            pl.BlockSpec((1, w), lambda i: (0, 0)),
        ],
        out_specs=pl.BlockSpec((blk, w), lambda i: (i, 0)),
        out_shape=jax.ShapeDtypeStruct((n, w), jnp.float32),
    )(mvc, wcat, bcat)


def _edge_score_sc(table, idxp, idxq, w2r, b2b, n_edges, hidden):
    ew = n_edges // _NW
    n_chunks = ew // _CE
    blocks = _CE // _L

    mesh = plsc.VectorSubcoreMesh(core_axis_name="c", subcore_axis_name="s")

    @pl.kernel(
        out_type=[
            jax.ShapeDtypeStruct((n_edges,), jnp.float32),
            jax.ShapeDtypeStruct((n_edges,), jnp.float32),
        ],
        mesh=mesh,
        compiler_params=pltpu.CompilerParams(
            use_tc_tiling_on_sc=False, needs_layout_passes=False
        ),
        scratch_types=[
            pltpu.VMEM((2, _CG, _GE), jnp.int32),
            pltpu.VMEM((2, _CG, _GE), jnp.int32),
            pltpu.VMEM((_CE, hidden), jnp.float32),
            pltpu.VMEM((_CE, hidden), jnp.float32),
            pltpu.VMEM((_CE, hidden), jnp.float32),
            pltpu.VMEM((_CE, hidden), jnp.float32),
            pltpu.VMEM((2, _CE), jnp.float32),
            pltpu.VMEM((2, _CE), jnp.float32),
            pltpu.VMEM((2, hidden, _L), jnp.float32),
            pltpu.VMEM((2, _L), jnp.float32),
            pltpu.SemaphoreType.DMA,
            pltpu.SemaphoreType.DMA,
            pltpu.SemaphoreType.DMA,
        ],
    )
    def k(t_hbm, idxp_hbm, idxq_hbm, w2_hbm, b2_hbm, out0_hbm, out1_hbm,
          ip_v, iq_v, p_a, q_a, p_b, q_b, o_a, o_b, w2_v, b2_v,
          sem_g, sem_oa, sem_ob):
        wid = lax.axis_index("s") * _NC + lax.axis_index("c")
        pltpu.sync_copy(w2_hbm, w2_v)
        pltpu.sync_copy(b2_hbm, b2_v)
        iota = lax.iota(jnp.int32, _L)
        zero = jnp.zeros((_L,), jnp.int32)

        def fire(c, p_buf, q_buf, slot):
            pltpu.sync_copy(idxp_hbm.at[wid * n_chunks + c], ip_v.at[slot])
            pltpu.sync_copy(idxq_hbm.at[wid * n_chunks + c], iq_v.at[slot])
            for g in range(_CG):
                pltpu.async_copy(
                    t_hbm.at[ip_v.at[slot, g]],
                    p_buf.at[pl.ds(g * _GE, _GE)],
                    sem_g,
                )
                pltpu.async_copy(
                    t_hbm.at[iq_v.at[slot, g]],
                    q_buf.at[pl.ds(g * _GE, _GE)],
                    sem_g,
                )

        def drain_gathers(p_buf, q_buf):
            for g in range(_CG):
                for buf in (p_buf, q_buf):
                    pltpu.make_async_copy(
                        t_hbm.at[pl.ds(0, _GE)],
                        buf.at[pl.ds(g * _GE, _GE)],
                        sem_g,
                    ).wait()

        def drain_out(o_buf, sem):
            for half in range(2):
                pltpu.make_async_copy(
                    out0_hbm.at[pl.ds(0, _CE)],
                    o_buf.at[half],
                    sem,
                ).wait()

        def compute(c, p_buf, q_buf, o_buf, sem):
            def pair_of_blocks(bs):
                rows = [_L * b + iota for b in bs]
                acc = [[b2_v[0, :], jnp.zeros((_L,), jnp.float32),
                        b2_v[1, :], jnp.zeros((_L,), jnp.float32)]
                       for _ in bs]
                for j in range(hidden):
                    col = (iota + j) % hidden
                    w0 = w2_v[0, j, :]
                    w1 = w2_v[1, j, :]
                    par = j & 1
                    for t in range(len(bs)):
                        p = plsc.load_gather(p_buf, [rows[t], col])
                        q = plsc.load_gather(q_buf, [rows[t], col])
                        r = jnp.maximum(p + q, 0.0)
                        acc[t][par] = acc[t][par] + r * w0
                        acc[t][2 + par] = acc[t][2 + par] + r * w1
                for t, b in enumerate(bs):
                    o_buf[0, pl.ds(b * _L, _L)] = acc[t][0] + acc[t][1]
                    o_buf[1, pl.ds(b * _L, _L)] = acc[t][2] + acc[t][3]

            @plsc.parallel_loop(0, blocks - 1, 6)
            def block_body(b):
                pair_of_blocks([b + t for t in range(6)])

            del block_body
            pair_of_blocks([blocks - 1])
            base_e = wid * ew + c * _CE
            pltpu.async_copy(o_buf.at[0], out0_hbm.at[pl.ds(base_e, _CE)],
                             sem)
            pltpu.async_copy(o_buf.at[1], out1_hbm.at[pl.ds(base_e, _CE)],
                             sem)

        fire(0, p_a, q_a, 0)
        fire(1, p_b, q_b, 1)

        def pair_body(i, _):
            c = 2 * i
            drain_gathers(p_a, q_a)

            @pl.when(c >= 2)
            def _():
                drain_out(o_a, sem_oa)

            compute(c, p_a, q_a, o_a, sem_oa)
            fire(c + 2, p_a, q_a, 0)
            drain_gathers(p_b, q_b)

            @pl.when(c >= 2)
            def _():
                drain_out(o_b, sem_ob)

            compute(c + 1, p_b, q_b, o_b, sem_ob)

            @pl.when(c + 3 < n_chunks)
            def _():
                fire(c + 3, p_b, q_b, 1)

            return 0

        lax.fori_loop(0, (n_chunks - 1) // 2, pair_body, 0)
        drain_gathers(p_a, q_a)
        drain_out(o_a, sem_oa)
        compute(n_chunks - 1, p_a, q_a, o_a, sem_oa)
        drain_out(o_b, sem_ob)
        drain_out(o_a, sem_oa)

    return k(table, idxp, idxq, w2r, b2b)


def kernel(mvc, edge_index, slow_edge_mask, W1, b1, W2, b2):
    n_nodes, d_feat = mvc.shape
    n_edges = edge_index.shape[1]
    hidden = W1.shape[0]

    ei = jnp.where(~slow_edge_mask, edge_index, 0)

    n_tot = n_edges // _CE
    idxp = (2 * ei[0]).reshape(n_tot, _CG, _GE)
    idxq = (2 * ei[1] + 1).reshape(n_tot, _CG, _GE)

    wcat = jnp.concatenate([W1[:, :d_feat].T, W1[:, d_feat:].T], axis=1)
    bcat = jnp.concatenate([b1, jnp.zeros((hidden,), jnp.float32)])[None, :]
    r_nodes = _node_projection(mvc, wcat, bcat)
    table = r_nodes.reshape(2 * n_nodes, hidden)

    jr = (jnp.arange(hidden)[:, None] + jnp.arange(_L)[None, :]) % hidden
    w2r = W2[:, jr].astype(jnp.float32)
    b2b = jnp.broadcast_to(b2[:, None], (2, _L)).astype(jnp.float32)

    out0, out1 = _edge_score_sc(table, idxp, idxq, w2r, b2b, n_edges, hidden)
    return jnp.stack([out0, out1], axis=1)


# speedup vs baseline: 1.1704x; 1.1704x over previous
import jax
import jax.numpy as jnp
from jax import lax
from jax.experimental import pallas as pl
from jax.experimental.pallas import tpu as pltpu
from jax.experimental.pallas import tpu_sc as plsc

_NC = 2
_NS = 16
_NW = _NC * _NS
_L = 16

_GE = 80
_CG = 5
_CE = _GE * _CG


def _proj_body(mvc_ref, w_ref, b_ref, out_ref):
    out_ref[...] = (
        jnp.dot(mvc_ref[...], w_ref[...], preferred_element_type=jnp.float32)
        + b_ref[...]
    )


def _node_projection(mvc, wcat, bcat):
    n, d = mvc.shape
    w = wcat.shape[1]
    blk = 1000
    return pl.pallas_call(
        _proj_body,
        grid=(n // blk,),
        in_specs=[
            pl.BlockSpec((blk, d), lambda i: (i, 0)),
            pl.BlockSpec((d, w), lambda i: (0, 0)),
            pl.BlockSpec((1, w), lambda i: (0, 0)),
        ],
        out_specs=pl.BlockSpec((blk, w), lambda i: (i, 0)),
        out_shape=jax.ShapeDtypeStruct((n, w), jnp.float32),
    )(mvc, wcat, bcat)


def _edge_score_sc(table, idxp, idxq, w2r, b2b, n_edges, hidden):
    ew = n_edges // _NW
    n_chunks = ew // _CE
    blocks = _CE // _L

    mesh = plsc.VectorSubcoreMesh(core_axis_name="c", subcore_axis_name="s")

    @pl.kernel(
        out_type=[
            jax.ShapeDtypeStruct((n_edges,), jnp.float32),
            jax.ShapeDtypeStruct((n_edges,), jnp.float32),
        ],
        mesh=mesh,
        compiler_params=pltpu.CompilerParams(
            use_tc_tiling_on_sc=False, needs_layout_passes=False
        ),
        scratch_types=[
            pltpu.VMEM((2, _CG, _GE), jnp.int32),
            pltpu.VMEM((2, _CG, _GE), jnp.int32),
            pltpu.VMEM((_CE, hidden), jnp.float32),
            pltpu.VMEM((_CE, hidden), jnp.float32),
            pltpu.VMEM((_CE, hidden), jnp.float32),
            pltpu.VMEM((_CE, hidden), jnp.float32),
            pltpu.VMEM((2, _CE), jnp.float32),
            pltpu.VMEM((2, _CE), jnp.float32),
            pltpu.VMEM((2, hidden, _L), jnp.float32),
            pltpu.VMEM((2, _L), jnp.float32),
            pltpu.SemaphoreType.DMA,
            pltpu.SemaphoreType.DMA,
            pltpu.SemaphoreType.DMA,
        ],
    )
    def k(t_hbm, idxp_hbm, idxq_hbm, w2_hbm, b2_hbm, out0_hbm, out1_hbm,
          ip_v, iq_v, p_a, q_a, p_b, q_b, o_a, o_b, w2_v, b2_v,
          sem_g, sem_oa, sem_ob):
        wid = lax.axis_index("s") * _NC + lax.axis_index("c")
        pltpu.sync_copy(w2_hbm, w2_v)
        pltpu.sync_copy(b2_hbm, b2_v)
        iota = lax.iota(jnp.int32, _L)
        zero = jnp.zeros((_L,), jnp.int32)

        def fire(c, p_buf, q_buf, slot):
            pltpu.sync_copy(idxp_hbm.at[wid * n_chunks + c], ip_v.at[slot])
            pltpu.sync_copy(idxq_hbm.at[wid * n_chunks + c], iq_v.at[slot])
            for g in range(_CG):
                pltpu.async_copy(
                    t_hbm.at[ip_v.at[slot, g]],
                    p_buf.at[pl.ds(g * _GE, _GE)],
                    sem_g,
                )
                pltpu.async_copy(
                    t_hbm.at[iq_v.at[slot, g]],
                    q_buf.at[pl.ds(g * _GE, _GE)],
                    sem_g,
                )

        def drain_gathers(p_buf, q_buf):
            for g in range(_CG):
                for buf in (p_buf, q_buf):
                    pltpu.make_async_copy(
                        t_hbm.at[pl.ds(0, _GE)],
                        buf.at[pl.ds(g * _GE, _GE)],
                        sem_g,
                    ).wait()

        def drain_out(o_buf, sem):
            for half in range(2):
                pltpu.make_async_copy(
                    out0_hbm.at[pl.ds(0, _CE)],
                    o_buf.at[half],
                    sem,
                ).wait()

        def compute(c, p_buf, q_buf, o_buf, sem):
            def pair_of_blocks(bs):
                rows = [_L * b + iota for b in bs]
                acc = [[b2_v[0, :], jnp.zeros((_L,), jnp.float32),
                        b2_v[1, :], jnp.zeros((_L,), jnp.float32)]
                       for _ in bs]
                for j in range(hidden):
                    col = (iota + j) % hidden
                    w0 = w2_v[0, j, :]
                    w1 = w2_v[1, j, :]
                    par = j & 1
                    for t in range(len(bs)):
                        p = plsc.load_gather(p_buf, [rows[t], col])
                        q = plsc.load_gather(q_buf, [rows[t], col])
                        r = jnp.maximum(p + q, 0.0)
                        acc[t][par] = acc[t][par] + r * w0
                        acc[t][2 + par] = acc[t][2 + par] + r * w1
                for t, b in enumerate(bs):
                    o_buf[0, pl.ds(b * _L, _L)] = acc[t][0] + acc[t][1]
                    o_buf[1, pl.ds(b * _L, _L)] = acc[t][2] + acc[t][3]

            @plsc.parallel_loop(0, blocks, 5)
            def block_body(b):
                pair_of_blocks([b + t for t in range(5)])

            del block_body
            base_e = wid * ew + c * _CE
            pltpu.async_copy(o_buf.at[0], out0_hbm.at[pl.ds(base_e, _CE)],
                             sem)
            pltpu.async_copy(o_buf.at[1], out1_hbm.at[pl.ds(base_e, _CE)],
                             sem)

        fire(0, p_a, q_a, 0)
        fire(1, p_b, q_b, 1)

        def pair_body(i, _):
            c = 2 * i
            drain_gathers(p_a, q_a)

            @pl.when(c >= 2)
            def _():
                drain_out(o_a, sem_oa)

            compute(c, p_a, q_a, o_a, sem_oa)
            fire(c + 2, p_a, q_a, 0)
            drain_gathers(p_b, q_b)

            @pl.when(c >= 2)
            def _():
                drain_out(o_b, sem_ob)

            compute(c + 1, p_b, q_b, o_b, sem_ob)

            @pl.when(c + 3 < n_chunks)
            def _():
                fire(c + 3, p_b, q_b, 1)

            return 0

        lax.fori_loop(0, (n_chunks - 1) // 2, pair_body, 0)
        drain_gathers(p_a, q_a)
        drain_out(o_a, sem_oa)
        compute(n_chunks - 1, p_a, q_a, o_a, sem_oa)
        drain_out(o_b, sem_ob)
        drain_out(o_a, sem_oa)

    return k(table, idxp, idxq, w2r, b2b)


def kernel(mvc, edge_index, slow_edge_mask, W1, b1, W2, b2):
    n_nodes, d_feat = mvc.shape
    n_edges = edge_index.shape[1]
    hidden = W1.shape[0]

    ei = jnp.where(~slow_edge_mask, edge_index, 0)

    n_tot = n_edges // _CE
    idxp = (2 * ei[0]).reshape(n_tot, _CG, _GE)
    idxq = (2 * ei[1] + 1).reshape(n_tot, _CG, _GE)

    wcat = jnp.concatenate([W1[:, :d_feat].T, W1[:, d_feat:].T], axis=1)
    bcat = jnp.concatenate([b1, jnp.zeros((hidden,), jnp.float32)])[None, :]
    r_nodes = _node_projection(mvc, wcat, bcat)
    table = r_nodes.reshape(2 * n_nodes, hidden)

    jr = (jnp.arange(hidden)[:, None] + jnp.arange(_L)[None, :]) % hidden
    w2r = W2[:, jr].astype(jnp.float32)
    b2b = jnp.broadcast_to(b2[:, None], (2, _L)).astype(jnp.float32)

    out0, out1 = _edge_score_sc(table, idxp, idxq, w2r, b2b, n_edges, hidden)
    return jnp.stack([out0, out1], axis=1)
